# baseline (speedup 1.0000x reference)
import functools

import jax
import jax.numpy as jnp
from jax import lax
from jax.experimental import pallas as pl
from jax.experimental.pallas import tpu as pltpu
from jax.experimental.pallas import tpu_sc as plsc

N = 10000
E = 320000
D = 128
DH = D // 2

NC = 2
NS = 16
NW = NC * NS
EPW = E // NW
B = 80
NCHUNK = EPW // B
NBUF = 4
ROUNDS = NCHUNK // NBUF
N_PAD = 10240
RPT = N_PAD // NS
ZR = 128
DW = 16

RB = 1000
GRID = N // RB

_MESH = dict(core_axis_name="c", subcore_axis_name="s")


def _edge_degree(dst3):
    @functools.partial(
        pl.kernel,
        out_type=jax.ShapeDtypeStruct((NC, N_PAD, DW), jnp.float32),
        mesh=plsc.VectorSubcoreMesh(**_MESH),
        scratch_types=[
            pltpu.VMEM((NCHUNK, B), jnp.int32),
            pltpu.VMEM((B, DW), jnp.float32),
            pltpu.VMEM((ZR, DW), jnp.float32),
            pltpu.VMEM_SHARED((N_PAD, DW), jnp.float32),
        ],
        compiler_params=pltpu.CompilerParams(use_tc_tiling_on_sc=False),
    )
    def k(dst_hbm, out_hbm, didx, ones, zbuf, acc):
        c = lax.axis_index("c")
        s = lax.axis_index("s")
        w = s * NC + c
        row0 = pl.multiple_of(s * RPT, 8)
        ov = jnp.ones((16,), jnp.float32)
        zv = jnp.zeros((16,), jnp.float32)

        def orow(i, t):
            ones[i, pl.ds(0, 16)] = ov
            return t

        lax.fori_loop(0, B, orow, 0)

        def zrow(i, t):
            zbuf[i, pl.ds(0, 16)] = zv
            return t

        lax.fori_loop(0, ZR, zrow, 0)
        for kk in range(RPT // ZR):
            pltpu.sync_copy(zbuf, acc.at[pl.ds(pl.multiple_of(row0 + kk * ZR, 8), ZR)])
        pltpu.sync_copy(dst_hbm.at[w], didx)
        plsc.subcore_barrier()

        def body(i, t):
            pltpu.sync_copy(ones, acc.at[didx.at[i]], add=True)
            return t

        lax.fori_loop(0, NCHUNK, body, 0)
        plsc.subcore_barrier()
        pltpu.sync_copy(acc.at[pl.ds(row0, RPT)],
                        out_hbm.at[c].at[pl.ds(row0, RPT)])

    return k(dst3)


def _edge_aggregate(g2h, src3, dst3):
    @functools.partial(
        pl.kernel,
        out_type=jax.ShapeDtypeStruct((NC, 2, N_PAD, DH), jnp.float32),
        mesh=plsc.VectorSubcoreMesh(**_MESH),
        scratch_types=[
            pltpu.VMEM((NCHUNK, B), jnp.int32),
            pltpu.VMEM((NCHUNK, B), jnp.int32),
            [pltpu.VMEM((B, DH), jnp.float32)] * NBUF,
            pltpu.VMEM((ZR, DH), jnp.float32),
            pltpu.VMEM_SHARED((N_PAD, DH), jnp.float32),
            [pltpu.SemaphoreType.DMA] * NBUF,
        ],
        compiler_params=pltpu.CompilerParams(use_tc_tiling_on_sc=False),
    )
    def k(g_hbm, src_hbm, dst_hbm, out_hbm, sidx, didx, rows, zbuf, acc, sems):
        c = lax.axis_index("c")
        s = lax.axis_index("s")
        w = s * NC + c
        row0 = pl.multiple_of(s * RPT, 8)
        zv = jnp.zeros((16,), jnp.float32)

        def zrow(i, t):
            for j in range(DH // 16):
                zbuf[i, pl.ds(j * 16, 16)] = zv
            return t

        lax.fori_loop(0, ZR, zrow, 0)
        pltpu.sync_copy(src_hbm.at[w], sidx)
        pltpu.sync_copy(dst_hbm.at[w], didx)

        for h in range(2):
            for kk in range(RPT // ZR):
                pltpu.sync_copy(
                    zbuf, acc.at[pl.ds(pl.multiple_of(row0 + kk * ZR, 8), ZR)])
            plsc.subcore_barrier()

            def body(r, t):
                base = r * NBUF
                descs = [
                    pltpu.async_copy(g_hbm.at[h].at[sidx.at[base + j]],
                                     rows[j], sems[j])
                    for j in range(NBUF)
                ]
                for j in range(NBUF):
                    descs[j].wait()
                for j in range(NBUF):
                    pltpu.sync_copy(rows[j], acc.at[didx.at[base + j]],
                                    add=True)
                return t

            lax.fori_loop(0, ROUNDS, body, 0)
            for i in range(NBUF * ROUNDS, NCHUNK):
                pltpu.async_copy(g_hbm.at[h].at[sidx.at[i]],
                                 rows[0], sems[0]).wait()
                pltpu.sync_copy(rows[0], acc.at[didx.at[i]], add=True)
            plsc.subcore_barrier()
            pltpu.sync_copy(acc.at[pl.ds(row0, RPT)],
                            out_hbm.at[c].at[h].at[pl.ds(row0, RPT)])

    return k(g2h, src3, dst3)


def _tc_scale1(x, W1, degp, interpret=False):
    def body(x_ref, w_ref, dp_ref, dis_ref, g1_ref):
        deg = dp_ref[0, :, 0] + dp_ref[1, :, 0] + 1.0
        dis = lax.rsqrt(deg)[:, None]
        h = jnp.dot(x_ref[...], w_ref[...], preferred_element_type=jnp.float32)
        g = h * dis
        dis_ref[...] = dis
        g1_ref[0] = g[:, :DH]
        g1_ref[1] = g[:, DH:]

    return pl.pallas_call(
        body,
        grid=(GRID,),
        in_specs=[
            pl.BlockSpec((RB, D), lambda i: (i, 0)),
            pl.BlockSpec((D, D), lambda i: (0, 0)),
            pl.BlockSpec((NC, RB, DW), lambda i: (0, i, 0)),
        ],
        out_specs=[
            pl.BlockSpec((RB, 1), lambda i: (i, 0)),
            pl.BlockSpec((2, RB, DH), lambda i: (0, i, 0)),
        ],
        out_shape=[
            jax.ShapeDtypeStruct((N, 1), jnp.float32),
            jax.ShapeDtypeStruct((2, N, DH), jnp.float32),
        ],
        interpret=interpret,
    )(x, W1, degp)


def _tc_layer2(p, g1, dis, xroot, W2, b1h, interpret=False):
    def body(p_ref, g1_ref, dis_ref, xr_ref, w2_ref, b1_ref, g2_ref):
        dis = dis_ref[...]
        rel = [
            jnp.maximum(
                dis * (p_ref[0, h] + p_ref[1, h] + g1_ref[h]) + b1_ref[h], 0.0)
            for h in range(2)
        ]
        r2 = jnp.dot(jnp.maximum(xr_ref[...], 0.0), w2_ref[D:, :],
                     preferred_element_type=jnp.float32)
        h2 = (jnp.dot(rel[0], w2_ref[:DH, :],
                      preferred_element_type=jnp.float32)
              + jnp.dot(rel[1], w2_ref[DH:D, :],
                        preferred_element_type=jnp.float32) + r2)
        g = dis * h2
        g2_ref[0] = g[:, :DH]
        g2_ref[1] = g[:, DH:]

    return pl.pallas_call(
        body,
        grid=(GRID,),
        in_specs=[
            pl.BlockSpec((NC, 2, RB, DH), lambda i: (0, 0, i, 0)),
            pl.BlockSpec((2, RB, DH), lambda i: (0, i, 0)),
            pl.BlockSpec((RB, 1), lambda i: (i, 0)),
            pl.BlockSpec((1, D), lambda i: (0, 0)),
            pl.BlockSpec((2 * D, D), lambda i: (0, 0)),
            pl.BlockSpec((2, 1, DH), lambda i: (0, 0, 0)),
        ],
        out_specs=pl.BlockSpec((2, RB, DH), lambda i: (0, i, 0)),
        out_shape=jax.ShapeDtypeStruct((2, N, DH), jnp.float32),
        interpret=interpret,
    )(p, g1, dis, xroot, W2, b1h)


def _tc_final(q, g2, dis, b2h, proots, g1r, disr, b1h, interpret=False):
    def body(q_ref, g2_ref, dis_ref, b2_ref, pr_ref, g1r_ref, disr_ref,
             b1_ref, out_ref):
        i = pl.program_id(0)
        dis = dis_ref[...]
        parts = []
        for h in range(2):
            conv2 = jnp.maximum(
                dis * (q_ref[0, h] + q_ref[1, h] + g2_ref[h]) + b2_ref[h], 0.0)
            parts.append(jnp.sum(conv2, axis=0, keepdims=True) * (1.0 / N))

        @pl.when(i == 0)
        def _():
            c1r = [
                (disr_ref[...] * (pr_ref[0, h] + pr_ref[1, h] + g1r_ref[h])
                 + b1_ref[h]) for h in range(2)
            ]
            out_ref[...] = jnp.concatenate(
                [c1r[0], c1r[1], parts[0], parts[1]], axis=1)

        @pl.when(i > 0)
        def _():
            zrow = jnp.zeros((1, D), jnp.float32)
            out_ref[...] += jnp.concatenate([zrow, parts[0], parts[1]], axis=1)

    return pl.pallas_call(
        body,
        grid=(GRID,),
        in_specs=[
            pl.BlockSpec((NC, 2, RB, DH), lambda i: (0, 0, i, 0)),
            pl.BlockSpec((2, RB, DH), lambda i: (0, i, 0)),
            pl.BlockSpec((RB, 1), lambda i: (i, 0)),
            pl.BlockSpec((2, 1, DH), lambda i: (0, 0, 0)),
            pl.BlockSpec((NC, 2, 1, DH), lambda i: (0, 0, 0, 0)),
            pl.BlockSpec((2, 1, DH), lambda i: (0, 0, 0)),
            pl.BlockSpec((1, 1), lambda i: (0, 0)),
            pl.BlockSpec((2, 1, DH), lambda i: (0, 0, 0)),
        ],
        out_specs=pl.BlockSpec((1, 2 * D), lambda i: (0, 0)),
        out_shape=jax.ShapeDtypeStruct((1, 2 * D), jnp.float32),
        interpret=interpret,
    )(q, g2, dis, b2h, proots, g1r, disr, b1h)


def kernel(x, edge_index, rootIndex, W1, b1, W2, b2):
    x = x.astype(jnp.float32)
    ei = edge_index.astype(jnp.int32)
    src3 = ei[0].reshape(NW, NCHUNK, B)
    dst3 = ei[1].reshape(NW, NCHUNK, B)
    r = jnp.asarray(rootIndex, jnp.int32)
    z = jnp.zeros((), jnp.int32)
    b1h = b1.reshape(2, 1, DH)
    b2h = b2.reshape(2, 1, DH)

    degp = _edge_degree(dst3)
    dis, g1 = _tc_scale1(x, W1, degp)
    p = _edge_aggregate(g1, src3, dst3)
    xroot = lax.dynamic_slice_in_dim(x, r, 1, axis=0)
    g2 = _tc_layer2(p, g1, dis, xroot, W2, b1h)
    q = _edge_aggregate(g2, src3, dst3)

    proots = lax.dynamic_slice(p, (z, z, r, z), (NC, 2, 1, DH))
    g1r = lax.dynamic_slice(g1, (z, r, z), (2, 1, DH))
    disr = lax.dynamic_slice(dis, (r, z), (1, 1))
    return _tc_final(q, g2, dis, b2h, proots, g1r, disr, b1h)


# speedup vs baseline: 1.1059x; 1.1059x over previous
import functools

import jax
import jax.numpy as jnp
from jax import lax
from jax.experimental import pallas as pl
from jax.experimental.pallas import tpu as pltpu
from jax.experimental.pallas import tpu_sc as plsc

N = 10000
E = 320000
D = 128
DH = D // 2

NC = 2
NS = 16
NW = NC * NS
EPW = E // NW
B = 80
NCHUNK = EPW // B
NBUF = 8
ROUNDS = NCHUNK // NBUF
N_PAD = 10240
RPT = N_PAD // NS
ZR = 128
DW = 16

RB = 1000
GRID = N // RB

_MESH = dict(core_axis_name="c", subcore_axis_name="s")


def _edge_degree(dst3):
    @functools.partial(
        pl.kernel,
        out_type=jax.ShapeDtypeStruct((NC, N_PAD, DW), jnp.float32),
        mesh=plsc.VectorSubcoreMesh(**_MESH),
        scratch_types=[
            pltpu.VMEM((NCHUNK, B), jnp.int32),
            pltpu.VMEM((B, DW), jnp.float32),
            pltpu.VMEM((ZR, DW), jnp.float32),
            pltpu.VMEM_SHARED((N_PAD, DW), jnp.float32),
        ],
        compiler_params=pltpu.CompilerParams(use_tc_tiling_on_sc=False),
    )
    def k(dst_hbm, out_hbm, didx, ones, zbuf, acc):
        c = lax.axis_index("c")
        s = lax.axis_index("s")
        w = s * NC + c
        row0 = pl.multiple_of(s * RPT, 8)
        ov = jnp.ones((16,), jnp.float32)
        zv = jnp.zeros((16,), jnp.float32)

        def orow(i, t):
            ones[i, pl.ds(0, 16)] = ov
            return t

        lax.fori_loop(0, B, orow, 0)

        def zrow(i, t):
            zbuf[i, pl.ds(0, 16)] = zv
            return t

        lax.fori_loop(0, ZR, zrow, 0)
        for kk in range(RPT // ZR):
            pltpu.sync_copy(zbuf, acc.at[pl.ds(pl.multiple_of(row0 + kk * ZR, 8), ZR)])
        pltpu.sync_copy(dst_hbm.at[w], didx)
        plsc.subcore_barrier()

        def body(i, t):
            pltpu.sync_copy(ones, acc.at[didx.at[i]], add=True)
            return t

        lax.fori_loop(0, NCHUNK, body, 0)
        plsc.subcore_barrier()
        pltpu.sync_copy(acc.at[pl.ds(row0, RPT)],
                        out_hbm.at[c].at[pl.ds(row0, RPT)])

    return k(dst3)


def _edge_aggregate(g2h, src3, dst3):
    @functools.partial(
        pl.kernel,
        out_type=jax.ShapeDtypeStruct((NC, 2, N_PAD, DH), jnp.float32),
        mesh=plsc.VectorSubcoreMesh(**_MESH),
        scratch_types=[
            pltpu.VMEM((NCHUNK, B), jnp.int32),
            pltpu.VMEM((NCHUNK, B), jnp.int32),
            [pltpu.VMEM((B, DH), jnp.float32)] * NBUF,
            pltpu.VMEM((ZR, DH), jnp.float32),
            pltpu.VMEM_SHARED((N_PAD, DH), jnp.float32),
            [pltpu.SemaphoreType.DMA] * NBUF,
        ],
        compiler_params=pltpu.CompilerParams(use_tc_tiling_on_sc=False),
    )
    def k(g_hbm, src_hbm, dst_hbm, out_hbm, sidx, didx, rows, zbuf, acc, sems):
        c = lax.axis_index("c")
        s = lax.axis_index("s")
        w = s * NC + c
        row0 = pl.multiple_of(s * RPT, 8)
        zv = jnp.zeros((16,), jnp.float32)

        def zrow(i, t):
            for j in range(DH // 16):
                zbuf[i, pl.ds(j * 16, 16)] = zv
            return t

        lax.fori_loop(0, ZR, zrow, 0)
        pltpu.sync_copy(src_hbm.at[w], sidx)
        pltpu.sync_copy(dst_hbm.at[w], didx)

        for h in range(2):
            for kk in range(RPT // ZR):
                pltpu.sync_copy(
                    zbuf, acc.at[pl.ds(pl.multiple_of(row0 + kk * ZR, 8), ZR)])
            plsc.subcore_barrier()

            def do_batch(base, cnt):
                descs = [
                    pltpu.async_copy(g_hbm.at[h].at[sidx.at[base + j]],
                                     rows[j], sems[j])
                    for j in range(cnt)
                ]
                for d in descs:
                    d.wait()
                sdescs = [
                    pltpu.async_copy(rows[j], acc.at[didx.at[base + j]],
                                     sems[j], add=True)
                    for j in range(cnt)
                ]
                for d in sdescs:
                    d.wait()

            def body(r, t):
                do_batch(r * NBUF, NBUF)
                return t

            lax.fori_loop(0, ROUNDS, body, 0)
            if NCHUNK % NBUF:
                do_batch(ROUNDS * NBUF, NCHUNK % NBUF)
            plsc.subcore_barrier()
            pltpu.sync_copy(acc.at[pl.ds(row0, RPT)],
                            out_hbm.at[c].at[h].at[pl.ds(row0, RPT)])

    return k(g2h, src3, dst3)


def _tc_scale1(x, W1, degp, interpret=False):
    def body(x_ref, w_ref, dp_ref, dis_ref, g1_ref):
        deg = dp_ref[0, :, 0] + dp_ref[1, :, 0] + 1.0
        dis = lax.rsqrt(deg)[:, None]
        h = jnp.dot(x_ref[...], w_ref[...], preferred_element_type=jnp.float32)
        g = h * dis
        dis_ref[...] = dis
        g1_ref[0] = g[:, :DH]
        g1_ref[1] = g[:, DH:]

    return pl.pallas_call(
        body,
        grid=(GRID,),
        in_specs=[
            pl.BlockSpec((RB, D), lambda i: (i, 0)),
            pl.BlockSpec((D, D), lambda i: (0, 0)),
            pl.BlockSpec((NC, RB, DW), lambda i: (0, i, 0)),
        ],
        out_specs=[
            pl.BlockSpec((RB, 1), lambda i: (i, 0)),
            pl.BlockSpec((2, RB, DH), lambda i: (0, i, 0)),
        ],
        out_shape=[
            jax.ShapeDtypeStruct((N, 1), jnp.float32),
            jax.ShapeDtypeStruct((2, N, DH), jnp.float32),
        ],
        interpret=interpret,
    )(x, W1, degp)


def _tc_layer2(p, g1, dis, xroot, W2, b1h, interpret=False):
    def body(p_ref, g1_ref, dis_ref, xr_ref, w2_ref, b1_ref, g2_ref):
        dis = dis_ref[...]
        rel = [
            jnp.maximum(
                dis * (p_ref[0, h] + p_ref[1, h] + g1_ref[h]) + b1_ref[h], 0.0)
            for h in range(2)
        ]
        r2 = jnp.dot(jnp.maximum(xr_ref[...], 0.0), w2_ref[D:, :],
                     preferred_element_type=jnp.float32)
        h2 = (jnp.dot(rel[0], w2_ref[:DH, :],
                      preferred_element_type=jnp.float32)
              + jnp.dot(rel[1], w2_ref[DH:D, :],
                        preferred_element_type=jnp.float32) + r2)
        g = dis * h2
        g2_ref[0] = g[:, :DH]
        g2_ref[1] = g[:, DH:]

    return pl.pallas_call(
        body,
        grid=(GRID,),
        in_specs=[
            pl.BlockSpec((NC, 2, RB, DH), lambda i: (0, 0, i, 0)),
            pl.BlockSpec((2, RB, DH), lambda i: (0, i, 0)),
            pl.BlockSpec((RB, 1), lambda i: (i, 0)),
            pl.BlockSpec((1, D), lambda i: (0, 0)),
            pl.BlockSpec((2 * D, D), lambda i: (0, 0)),
            pl.BlockSpec((2, 1, DH), lambda i: (0, 0, 0)),
        ],
        out_specs=pl.BlockSpec((2, RB, DH), lambda i: (0, i, 0)),
        out_shape=jax.ShapeDtypeStruct((2, N, DH), jnp.float32),
        interpret=interpret,
    )(p, g1, dis, xroot, W2, b1h)


def _tc_final(q, g2, dis, b2h, proots, g1r, disr, b1h, interpret=False):
    def body(q_ref, g2_ref, dis_ref, b2_ref, pr_ref, g1r_ref, disr_ref,
             b1_ref, out_ref):
        i = pl.program_id(0)
        dis = dis_ref[...]
        parts = []
        for h in range(2):
            conv2 = jnp.maximum(
                dis * (q_ref[0, h] + q_ref[1, h] + g2_ref[h]) + b2_ref[h], 0.0)
            parts.append(jnp.sum(conv2, axis=0, keepdims=True) * (1.0 / N))

        @pl.when(i == 0)
        def _():
            c1r = [
                (disr_ref[...] * (pr_ref[0, h] + pr_ref[1, h] + g1r_ref[h])
                 + b1_ref[h]) for h in range(2)
            ]
            out_ref[...] = jnp.concatenate(
                [c1r[0], c1r[1], parts[0], parts[1]], axis=1)

        @pl.when(i > 0)
        def _():
            zrow = jnp.zeros((1, D), jnp.float32)
            out_ref[...] += jnp.concatenate([zrow, parts[0], parts[1]], axis=1)

    return pl.pallas_call(
        body,
        grid=(GRID,),
        in_specs=[
            pl.BlockSpec((NC, 2, RB, DH), lambda i: (0, 0, i, 0)),
            pl.BlockSpec((2, RB, DH), lambda i: (0, i, 0)),
            pl.BlockSpec((RB, 1), lambda i: (i, 0)),
            pl.BlockSpec((2, 1, DH), lambda i: (0, 0, 0)),
            pl.BlockSpec((NC, 2, 1, DH), lambda i: (0, 0, 0, 0)),
            pl.BlockSpec((2, 1, DH), lambda i: (0, 0, 0)),
            pl.BlockSpec((1, 1), lambda i: (0, 0)),
            pl.BlockSpec((2, 1, DH), lambda i: (0, 0, 0)),
        ],
        out_specs=pl.BlockSpec((1, 2 * D), lambda i: (0, 0)),
        out_shape=jax.ShapeDtypeStruct((1, 2 * D), jnp.float32),
        interpret=interpret,
    )(q, g2, dis, b2h, proots, g1r, disr, b1h)


def kernel(x, edge_index, rootIndex, W1, b1, W2, b2):
    x = x.astype(jnp.float32)
    ei = edge_index.astype(jnp.int32)
    src3 = ei[0].reshape(NW, NCHUNK, B)
    dst3 = ei[1].reshape(NW, NCHUNK, B)
    r = jnp.asarray(rootIndex, jnp.int32)
    z = jnp.zeros((), jnp.int32)
    b1h = b1.reshape(2, 1, DH)
    b2h = b2.reshape(2, 1, DH)

    degp = _edge_degree(dst3)
    dis, g1 = _tc_scale1(x, W1, degp)
    p = _edge_aggregate(g1, src3, dst3)
    xroot = lax.dynamic_slice_in_dim(x, r, 1, axis=0)
    g2 = _tc_layer2(p, g1, dis, xroot, W2, b1h)
    q = _edge_aggregate(g2, src3, dst3)

    proots = lax.dynamic_slice(p, (z, z, r, z), (NC, 2, 1, DH))
    g1r = lax.dynamic_slice(g1, (z, r, z), (2, 1, DH))
    disr = lax.dynamic_slice(dis, (r, z), (1, 1))
    return _tc_final(q, g2, dis, b2h, proots, g1r, disr, b1h)


# speedup vs baseline: 1.1220x; 1.0146x over previous
import functools

import jax
import jax.numpy as jnp
from jax import lax
from jax.experimental import pallas as pl
from jax.experimental.pallas import tpu as pltpu
from jax.experimental.pallas import tpu_sc as plsc

N = 10000
E = 320000
D = 128
DH = D // 2

NC = 2
NS = 16
NW = NC * NS
EPW = E // NW
B = 80
NCHUNK = EPW // B
NBUF = 12
ROUNDS = NCHUNK // NBUF
N_PAD = 10240
RPT = N_PAD // NS
ZR = 128
DW = 16

RB = 1000
GRID = N // RB

_MESH = dict(core_axis_name="c", subcore_axis_name="s")


def _edge_degree(dst3):
    @functools.partial(
        pl.kernel,
        out_type=jax.ShapeDtypeStruct((NC, N_PAD, DW), jnp.float32),
        mesh=plsc.VectorSubcoreMesh(**_MESH),
        scratch_types=[
            pltpu.VMEM((NCHUNK, B), jnp.int32),
            pltpu.VMEM((B, DW), jnp.float32),
            pltpu.VMEM((ZR, DW), jnp.float32),
            pltpu.VMEM_SHARED((N_PAD, DW), jnp.float32),
        ],
        compiler_params=pltpu.CompilerParams(use_tc_tiling_on_sc=False),
    )
    def k(dst_hbm, out_hbm, didx, ones, zbuf, acc):
        c = lax.axis_index("c")
        s = lax.axis_index("s")
        w = s * NC + c
        row0 = pl.multiple_of(s * RPT, 8)
        ov = jnp.ones((16,), jnp.float32)
        zv = jnp.zeros((16,), jnp.float32)

        def orow(i, t):
            ones[i, pl.ds(0, 16)] = ov
            return t

        lax.fori_loop(0, B, orow, 0)

        def zrow(i, t):
            zbuf[i, pl.ds(0, 16)] = zv
            return t

        lax.fori_loop(0, ZR, zrow, 0)
        for kk in range(RPT // ZR):
            pltpu.sync_copy(zbuf, acc.at[pl.ds(pl.multiple_of(row0 + kk * ZR, 8), ZR)])
        pltpu.sync_copy(dst_hbm.at[w], didx)
        plsc.subcore_barrier()

        def body(i, t):
            pltpu.sync_copy(ones, acc.at[didx.at[i]], add=True)
            return t

        lax.fori_loop(0, NCHUNK, body, 0)
        plsc.subcore_barrier()
        pltpu.sync_copy(acc.at[pl.ds(row0, RPT)],
                        out_hbm.at[c].at[pl.ds(row0, RPT)])

    return k(dst3)


def _edge_aggregate(g2h, src3, dst3):
    @functools.partial(
        pl.kernel,
        out_type=jax.ShapeDtypeStruct((NC, 2, N_PAD, DH), jnp.float32),
        mesh=plsc.VectorSubcoreMesh(**_MESH),
        scratch_types=[
            pltpu.VMEM((NCHUNK, B), jnp.int32),
            pltpu.VMEM((NCHUNK, B), jnp.int32),
            [pltpu.VMEM((B, DH), jnp.float32)] * NBUF,
            pltpu.VMEM((ZR, DH), jnp.float32),
            pltpu.VMEM_SHARED((N_PAD, DH), jnp.float32),
            [pltpu.SemaphoreType.DMA] * NBUF,
        ],
        compiler_params=pltpu.CompilerParams(use_tc_tiling_on_sc=False),
    )
    def k(g_hbm, src_hbm, dst_hbm, out_hbm, sidx, didx, rows, zbuf, acc, sems):
        c = lax.axis_index("c")
        s = lax.axis_index("s")
        w = s * NC + c
        row0 = pl.multiple_of(s * RPT, 8)
        zv = jnp.zeros((16,), jnp.float32)

        def zrow(i, t):
            for j in range(DH // 16):
                zbuf[i, pl.ds(j * 16, 16)] = zv
            return t

        lax.fori_loop(0, ZR, zrow, 0)
        pltpu.sync_copy(src_hbm.at[w], sidx)
        pltpu.sync_copy(dst_hbm.at[w], didx)

        for h in range(2):
            for kk in range(RPT // ZR):
                pltpu.sync_copy(
                    zbuf, acc.at[pl.ds(pl.multiple_of(row0 + kk * ZR, 8), ZR)])
            plsc.subcore_barrier()

            def do_batch(base, cnt):
                descs = [
                    pltpu.async_copy(g_hbm.at[h].at[sidx.at[base + j]],
                                     rows[j], sems[j])
                    for j in range(cnt)
                ]
                for d in descs:
                    d.wait()
                sdescs = [
                    pltpu.async_copy(rows[j], acc.at[didx.at[base + j]],
                                     sems[j], add=True)
                    for j in range(cnt)
                ]
                for d in sdescs:
                    d.wait()

            def body(r, t):
                do_batch(r * NBUF, NBUF)
                return t

            lax.fori_loop(0, ROUNDS, body, 0)
            if NCHUNK % NBUF:
                do_batch(ROUNDS * NBUF, NCHUNK % NBUF)
            plsc.subcore_barrier()
            pltpu.sync_copy(acc.at[pl.ds(row0, RPT)],
                            out_hbm.at[c].at[h].at[pl.ds(row0, RPT)])

    return k(g2h, src3, dst3)


def _tc_scale1(x, W1, degp, interpret=False):
    def body(x_ref, w_ref, dp_ref, dis_ref, g1_ref):
        deg = dp_ref[0, :, 0] + dp_ref[1, :, 0] + 1.0
        dis = lax.rsqrt(deg)[:, None]
        h = jnp.dot(x_ref[...], w_ref[...], preferred_element_type=jnp.float32)
        g = h * dis
        dis_ref[...] = dis
        g1_ref[0] = g[:, :DH]
        g1_ref[1] = g[:, DH:]

    return pl.pallas_call(
        body,
        grid=(GRID,),
        in_specs=[
            pl.BlockSpec((RB, D), lambda i: (i, 0)),
            pl.BlockSpec((D, D), lambda i: (0, 0)),
            pl.BlockSpec((NC, RB, DW), lambda i: (0, i, 0)),
        ],
        out_specs=[
            pl.BlockSpec((RB, 1), lambda i: (i, 0)),
            pl.BlockSpec((2, RB, DH), lambda i: (0, i, 0)),
        ],
        out_shape=[
            jax.ShapeDtypeStruct((N, 1), jnp.float32),
            jax.ShapeDtypeStruct((2, N, DH), jnp.float32),
        ],
        interpret=interpret,
    )(x, W1, degp)


def _tc_layer2(p, g1, dis, xroot, W2, b1h, interpret=False):
    def body(p_ref, g1_ref, dis_ref, xr_ref, w2_ref, b1_ref, g2_ref):
        dis = dis_ref[...]
        rel = [
            jnp.maximum(
                dis * (p_ref[0, h] + p_ref[1, h] + g1_ref[h]) + b1_ref[h], 0.0)
            for h in range(2)
        ]
        r2 = jnp.dot(jnp.maximum(xr_ref[...], 0.0), w2_ref[D:, :],
                     preferred_element_type=jnp.float32)
        h2 = (jnp.dot(rel[0], w2_ref[:DH, :],
                      preferred_element_type=jnp.float32)
              + jnp.dot(rel[1], w2_ref[DH:D, :],
                        preferred_element_type=jnp.float32) + r2)
        g = dis * h2
        g2_ref[0] = g[:, :DH]
        g2_ref[1] = g[:, DH:]

    return pl.pallas_call(
        body,
        grid=(GRID,),
        in_specs=[
            pl.BlockSpec((NC, 2, RB, DH), lambda i: (0, 0, i, 0)),
            pl.BlockSpec((2, RB, DH), lambda i: (0, i, 0)),
            pl.BlockSpec((RB, 1), lambda i: (i, 0)),
            pl.BlockSpec((1, D), lambda i: (0, 0)),
            pl.BlockSpec((2 * D, D), lambda i: (0, 0)),
            pl.BlockSpec((2, 1, DH), lambda i: (0, 0, 0)),
        ],
        out_specs=pl.BlockSpec((2, RB, DH), lambda i: (0, i, 0)),
        out_shape=jax.ShapeDtypeStruct((2, N, DH), jnp.float32),
        interpret=interpret,
    )(p, g1, dis, xroot, W2, b1h)


def _tc_final(q, g2, dis, b2h, proots, g1r, disr, b1h, interpret=False):
    def body(q_ref, g2_ref, dis_ref, b2_ref, pr_ref, g1r_ref, disr_ref,
             b1_ref, out_ref):
        i = pl.program_id(0)
        dis = dis_ref[...]
        parts = []
        for h in range(2):
            conv2 = jnp.maximum(
                dis * (q_ref[0, h] + q_ref[1, h] + g2_ref[h]) + b2_ref[h], 0.0)
            parts.append(jnp.sum(conv2, axis=0, keepdims=True) * (1.0 / N))

        @pl.when(i == 0)
        def _():
            c1r = [
                (disr_ref[...] * (pr_ref[0, h] + pr_ref[1, h] + g1r_ref[h])
                 + b1_ref[h]) for h in range(2)
            ]
            out_ref[...] = jnp.concatenate(
                [c1r[0], c1r[1], parts[0], parts[1]], axis=1)

        @pl.when(i > 0)
        def _():
            zrow = jnp.zeros((1, D), jnp.float32)
            out_ref[...] += jnp.concatenate([zrow, parts[0], parts[1]], axis=1)

    return pl.pallas_call(
        body,
        grid=(GRID,),
        in_specs=[
            pl.BlockSpec((NC, 2, RB, DH), lambda i: (0, 0, i, 0)),
            pl.BlockSpec((2, RB, DH), lambda i: (0, i, 0)),
            pl.BlockSpec((RB, 1), lambda i: (i, 0)),
            pl.BlockSpec((2, 1, DH), lambda i: (0, 0, 0)),
            pl.BlockSpec((NC, 2, 1, DH), lambda i: (0, 0, 0, 0)),
            pl.BlockSpec((2, 1, DH), lambda i: (0, 0, 0)),
            pl.BlockSpec((1, 1), lambda i: (0, 0)),
            pl.BlockSpec((2, 1, DH), lambda i: (0, 0, 0)),
        ],
        out_specs=pl.BlockSpec((1, 2 * D), lambda i: (0, 0)),
        out_shape=jax.ShapeDtypeStruct((1, 2 * D), jnp.float32),
        interpret=interpret,
    )(q, g2, dis, b2h, proots, g1r, disr, b1h)


def kernel(x, edge_index, rootIndex, W1, b1, W2, b2):
    x = x.astype(jnp.float32)
    ei = edge_index.astype(jnp.int32)
    src3 = ei[0].reshape(NW, NCHUNK, B)
    dst3 = ei[1].reshape(NW, NCHUNK, B)
    r = jnp.asarray(rootIndex, jnp.int32)
    z = jnp.zeros((), jnp.int32)
    b1h = b1.reshape(2, 1, DH)
    b2h = b2.reshape(2, 1, DH)

    degp = _edge_degree(dst3)
    dis, g1 = _tc_scale1(x, W1, degp)
    p = _edge_aggregate(g1, src3, dst3)
    xroot = lax.dynamic_slice_in_dim(x, r, 1, axis=0)
    g2 = _tc_layer2(p, g1, dis, xroot, W2, b1h)
    q = _edge_aggregate(g2, src3, dst3)

    proots = lax.dynamic_slice(p, (z, z, r, z), (NC, 2, 1, DH))
    g1r = lax.dynamic_slice(g1, (z, r, z), (2, 1, DH))
    disr = lax.dynamic_slice(dis, (r, z), (1, 1))
    return _tc_final(q, g2, dis, b2h, proots, g1r, disr, b1h)


# speedup vs baseline: 1.1379x; 1.0142x over previous
import functools

import jax
import jax.numpy as jnp
from jax import lax
from jax.experimental import pallas as pl
from jax.experimental.pallas import tpu as pltpu
from jax.experimental.pallas import tpu_sc as plsc

N = 10000
E = 320000
D = 128
DH = D // 2

NC = 2
NS = 16
NW = NC * NS
EPW = E // NW
B = 80
NCHUNK = EPW // B
NBUF = 12
ROUNDS = NCHUNK // NBUF
N_PAD = 10240
RPT = N_PAD // NS
ZR = 128
DW = 16

RB = 1000
GRID = N // RB

_MESH = dict(core_axis_name="c", subcore_axis_name="s")


def _edge_degree(dst3):
    @functools.partial(
        pl.kernel,
        out_type=jax.ShapeDtypeStruct((NC, N_PAD, DW), jnp.float32),
        mesh=plsc.VectorSubcoreMesh(**_MESH),
        scratch_types=[
            pltpu.VMEM((NCHUNK, B), jnp.int32),
            pltpu.VMEM((B, DW), jnp.float32),
            pltpu.VMEM((ZR, DW), jnp.float32),
            pltpu.VMEM_SHARED((N_PAD, DW), jnp.float32),
            [pltpu.SemaphoreType.DMA] * NBUF,
        ],
        compiler_params=pltpu.CompilerParams(use_tc_tiling_on_sc=False),
    )
    def k(dst_hbm, out_hbm, didx, ones, zbuf, acc, sems):
        c = lax.axis_index("c")
        s = lax.axis_index("s")
        w = s * NC + c
        row0 = pl.multiple_of(s * RPT, 8)
        ov = jnp.ones((16,), jnp.float32)
        zv = jnp.zeros((16,), jnp.float32)

        def orow(i, t):
            ones[i, pl.ds(0, 16)] = ov
            return t

        lax.fori_loop(0, B, orow, 0)

        def zrow(i, t):
            zbuf[i, pl.ds(0, 16)] = zv
            return t

        lax.fori_loop(0, ZR, zrow, 0)
        for kk in range(RPT // ZR):
            pltpu.sync_copy(zbuf, acc.at[pl.ds(pl.multiple_of(row0 + kk * ZR, 8), ZR)])
        pltpu.sync_copy(dst_hbm.at[w], didx)
        plsc.subcore_barrier()

        def do_batch(base, cnt):
            descs = [
                pltpu.async_copy(ones, acc.at[didx.at[base + j]], sems[j],
                                 add=True)
                for j in range(cnt)
            ]
            for d in descs:
                d.wait()

        def body(r, t):
            do_batch(r * NBUF, NBUF)
            return t

        lax.fori_loop(0, ROUNDS, body, 0)
        if NCHUNK % NBUF:
            do_batch(ROUNDS * NBUF, NCHUNK % NBUF)
        plsc.subcore_barrier()
        pltpu.sync_copy(acc.at[pl.ds(row0, RPT)],
                        out_hbm.at[c].at[pl.ds(row0, RPT)])

    return k(dst3)


def _edge_aggregate(g2h, src3, dst3):
    @functools.partial(
        pl.kernel,
        out_type=jax.ShapeDtypeStruct((NC, 2, N_PAD, DH), jnp.float32),
        mesh=plsc.VectorSubcoreMesh(**_MESH),
        scratch_types=[
            pltpu.VMEM((NCHUNK, B), jnp.int32),
            pltpu.VMEM((NCHUNK, B), jnp.int32),
            [pltpu.VMEM((B, DH), jnp.float32)] * NBUF,
            pltpu.VMEM((ZR, DH), jnp.float32),
            pltpu.VMEM_SHARED((N_PAD, DH), jnp.float32),
            [pltpu.SemaphoreType.DMA] * NBUF,
        ],
        compiler_params=pltpu.CompilerParams(use_tc_tiling_on_sc=False),
    )
    def k(g_hbm, src_hbm, dst_hbm, out_hbm, sidx, didx, rows, zbuf, acc, sems):
        c = lax.axis_index("c")
        s = lax.axis_index("s")
        w = s * NC + c
        row0 = pl.multiple_of(s * RPT, 8)
        zv = jnp.zeros((16,), jnp.float32)

        def zrow(i, t):
            for j in range(DH // 16):
                zbuf[i, pl.ds(j * 16, 16)] = zv
            return t

        lax.fori_loop(0, ZR, zrow, 0)
        pltpu.sync_copy(src_hbm.at[w], sidx)
        pltpu.sync_copy(dst_hbm.at[w], didx)

        for h in range(2):
            for kk in range(RPT // ZR):
                pltpu.sync_copy(
                    zbuf, acc.at[pl.ds(pl.multiple_of(row0 + kk * ZR, 8), ZR)])
            plsc.subcore_barrier()

            def do_batch(base, cnt):
                descs = [
                    pltpu.async_copy(g_hbm.at[h].at[sidx.at[base + j]],
                                     rows[j], sems[j])
                    for j in range(cnt)
                ]
                for d in descs:
                    d.wait()
                sdescs = [
                    pltpu.async_copy(rows[j], acc.at[didx.at[base + j]],
                                     sems[j], add=True)
                    for j in range(cnt)
                ]
                for d in sdescs:
                    d.wait()

            def body(r, t):
                do_batch(r * NBUF, NBUF)
                return t

            lax.fori_loop(0, ROUNDS, body, 0)
            if NCHUNK % NBUF:
                do_batch(ROUNDS * NBUF, NCHUNK % NBUF)
            plsc.subcore_barrier()
            pltpu.sync_copy(acc.at[pl.ds(row0, RPT)],
                            out_hbm.at[c].at[h].at[pl.ds(row0, RPT)])

    return k(g2h, src3, dst3)


def _tc_scale1(x, W1, degp, interpret=False):
    def body(x_ref, w_ref, dp_ref, dis_ref, g1_ref):
        deg = dp_ref[0, :, 0] + dp_ref[1, :, 0] + 1.0
        dis = lax.rsqrt(deg)[:, None]
        h = jnp.dot(x_ref[...], w_ref[...], preferred_element_type=jnp.float32)
        g = h * dis
        dis_ref[...] = dis
        g1_ref[0] = g[:, :DH]
        g1_ref[1] = g[:, DH:]

    return pl.pallas_call(
        body,
        grid=(GRID,),
        in_specs=[
            pl.BlockSpec((RB, D), lambda i: (i, 0)),
            pl.BlockSpec((D, D), lambda i: (0, 0)),
            pl.BlockSpec((NC, RB, DW), lambda i: (0, i, 0)),
        ],
        out_specs=[
            pl.BlockSpec((RB, 1), lambda i: (i, 0)),
            pl.BlockSpec((2, RB, DH), lambda i: (0, i, 0)),
        ],
        out_shape=[
            jax.ShapeDtypeStruct((N, 1), jnp.float32),
            jax.ShapeDtypeStruct((2, N, DH), jnp.float32),
        ],
        interpret=interpret,
    )(x, W1, degp)


def _tc_layer2(p, g1, dis, xroot, W2, b1h, interpret=False):
    def body(p_ref, g1_ref, dis_ref, xr_ref, w2_ref, b1_ref, g2_ref):
        dis = dis_ref[...]
        rel = [
            jnp.maximum(
                dis * (p_ref[0, h] + p_ref[1, h] + g1_ref[h]) + b1_ref[h], 0.0)
            for h in range(2)
        ]
        r2 = jnp.dot(jnp.maximum(xr_ref[...], 0.0), w2_ref[D:, :],
                     preferred_element_type=jnp.float32)
        h2 = (jnp.dot(rel[0], w2_ref[:DH, :],
                      preferred_element_type=jnp.float32)
              + jnp.dot(rel[1], w2_ref[DH:D, :],
                        preferred_element_type=jnp.float32) + r2)
        g = dis * h2
        g2_ref[0] = g[:, :DH]
        g2_ref[1] = g[:, DH:]

    return pl.pallas_call(
        body,
        grid=(GRID,),
        in_specs=[
            pl.BlockSpec((NC, 2, RB, DH), lambda i: (0, 0, i, 0)),
            pl.BlockSpec((2, RB, DH), lambda i: (0, i, 0)),
            pl.BlockSpec((RB, 1), lambda i: (i, 0)),
            pl.BlockSpec((1, D), lambda i: (0, 0)),
            pl.BlockSpec((2 * D, D), lambda i: (0, 0)),
            pl.BlockSpec((2, 1, DH), lambda i: (0, 0, 0)),
        ],
        out_specs=pl.BlockSpec((2, RB, DH), lambda i: (0, i, 0)),
        out_shape=jax.ShapeDtypeStruct((2, N, DH), jnp.float32),
        interpret=interpret,
    )(p, g1, dis, xroot, W2, b1h)


def _tc_final(q, g2, dis, b2h, proots, g1r, disr, b1h, interpret=False):
    def body(q_ref, g2_ref, dis_ref, b2_ref, pr_ref, g1r_ref, disr_ref,
             b1_ref, out_ref):
        i = pl.program_id(0)
        dis = dis_ref[...]
        parts = []
        for h in range(2):
            conv2 = jnp.maximum(
                dis * (q_ref[0, h] + q_ref[1, h] + g2_ref[h]) + b2_ref[h], 0.0)
            parts.append(jnp.sum(conv2, axis=0, keepdims=True) * (1.0 / N))

        @pl.when(i == 0)
        def _():
            c1r = [
                (disr_ref[...] * (pr_ref[0, h] + pr_ref[1, h] + g1r_ref[h])
                 + b1_ref[h]) for h in range(2)
            ]
            out_ref[...] = jnp.concatenate(
                [c1r[0], c1r[1], parts[0], parts[1]], axis=1)

        @pl.when(i > 0)
        def _():
            zrow = jnp.zeros((1, D), jnp.float32)
            out_ref[...] += jnp.concatenate([zrow, parts[0], parts[1]], axis=1)

    return pl.pallas_call(
        body,
        grid=(GRID,),
        in_specs=[
            pl.BlockSpec((NC, 2, RB, DH), lambda i: (0, 0, i, 0)),
            pl.BlockSpec((2, RB, DH), lambda i: (0, i, 0)),
            pl.BlockSpec((RB, 1), lambda i: (i, 0)),
            pl.BlockSpec((2, 1, DH), lambda i: (0, 0, 0)),
            pl.BlockSpec((NC, 2, 1, DH), lambda i: (0, 0, 0, 0)),
            pl.BlockSpec((2, 1, DH), lambda i: (0, 0, 0)),
            pl.BlockSpec((1, 1), lambda i: (0, 0)),
            pl.BlockSpec((2, 1, DH), lambda i: (0, 0, 0)),
        ],
        out_specs=pl.BlockSpec((1, 2 * D), lambda i: (0, 0)),
        out_shape=jax.ShapeDtypeStruct((1, 2 * D), jnp.float32),
        interpret=interpret,
    )(q, g2, dis, b2h, proots, g1r, disr, b1h)


def kernel(x, edge_index, rootIndex, W1, b1, W2, b2):
    x = x.astype(jnp.float32)
    ei = edge_index.astype(jnp.int32)
    src3 = ei[0].reshape(NW, NCHUNK, B)
    dst3 = ei[1].reshape(NW, NCHUNK, B)
    r = jnp.asarray(rootIndex, jnp.int32)
    z = jnp.zeros((), jnp.int32)
    b1h = b1.reshape(2, 1, DH)
    b2h = b2.reshape(2, 1, DH)

    degp = _edge_degree(dst3)
    dis, g1 = _tc_scale1(x, W1, degp)
    p = _edge_aggregate(g1, src3, dst3)
    xroot = lax.dynamic_slice_in_dim(x, r, 1, axis=0)
    g2 = _tc_layer2(p, g1, dis, xroot, W2, b1h)
    q = _edge_aggregate(g2, src3, dst3)

    proots = lax.dynamic_slice(p, (z, z, r, z), (NC, 2, 1, DH))
    g1r = lax.dynamic_slice(g1, (z, r, z), (2, 1, DH))
    disr = lax.dynamic_slice(dis, (r, z), (1, 1))
    return _tc_final(q, g2, dis, b2h, proots, g1r, disr, b1h)
